# baseline (speedup 1.0000x reference)
import functools

import jax
import jax.numpy as jnp
from jax import lax
from jax.experimental import pallas as pl
from jax.experimental.pallas import tpu as pltpu
from jax.experimental.pallas import tpu_sc as plsc

_VOCAB = 1000000
_D = 64
_MAX_LEN = 200
_BATCH = 4096
_B = _BATCH * _MAX_LEN

_NC = 2
_NS = 16
_NW = _NC * _NS
_PER_W = _B // _NW
_C = 200
_NCHUNK = _PER_W // _C


def _positional() -> jax.Array:
    position = jnp.arange(0, _MAX_LEN, dtype=jnp.float32).reshape(-1, 1)
    div = jnp.exp(
        jnp.arange(0, _D, 2, dtype=jnp.float32) / _D
        * -jnp.log(jnp.float32(10000.0))
    )
    p = jnp.zeros((_MAX_LEN, _D), dtype=jnp.float32)
    p = p.at[:, 0::2].set(jnp.sin(position * div))
    p = p.at[:, 1::2].set(jnp.cos(position * div))
    return p


_mesh = plsc.VectorSubcoreMesh(core_axis_name="c", subcore_axis_name="s")


@functools.partial(
    pl.kernel,
    mesh=_mesh,
    out_type=jax.ShapeDtypeStruct((_B, _D), jnp.float32),
    scratch_types=[
        pltpu.SMEM((3, _C), jnp.int32),
        pltpu.VMEM_SHARED((_NS, 3, _C), jnp.int32),
        pltpu.SemaphoreType.DMA((3,)),
        pltpu.SemaphoreType.DMA,
    ],
    compiler_params=pltpu.CompilerParams(use_tc_tiling_on_sc=False),
)
def _gather(x_hbm, table_hbm, out_hbm, idxs, idxv, isem, gsem):
    sid = lax.axis_index("s")
    wid = sid * _NC + lax.axis_index("c")
    base = wid * _PER_W

    def idx_load(k, slot):
        pltpu.async_copy(
            x_hbm.at[pl.ds(base + k * _C, _C)], idxv.at[sid].at[slot],
            isem.at[slot])

    def idx_wait(k, slot):
        pltpu.make_async_copy(
            x_hbm.at[pl.ds(base + k * _C, _C)], idxv.at[sid].at[slot],
            isem.at[slot]).wait()
        pltpu.sync_copy(idxv.at[sid].at[slot], idxs.at[slot])

    def fire_row(slot, k, r):
        idx = idxs[slot, r]
        pltpu.async_copy(
            table_hbm.at[pl.ds(idx, 1)],
            out_hbm.at[pl.ds(base + k * _C + r, 1)],
            gsem,
        )

    pltpu.sync_copy(x_hbm.at[pl.ds(base, _C)], idxv.at[sid].at[0])
    pltpu.sync_copy(idxv.at[sid].at[0], idxs.at[0])
    idx_load(1, 1)

    def chunk_body(k, carry):
        slot = lax.rem(k, 3)
        fb = lax.rem(k + 2, 3)

        @pl.when(k + 2 < _NCHUNK)
        def _fire_next_idx():
            idx_load(k + 2, fb)

        def body(r, c2):
            fire_row(slot, k, r)
            return c2

        lax.fori_loop(0, _C, body, 0)

        @pl.when(k + 1 < _NCHUNK)
        def _wait_next_idx():
            idx_wait(k + 1, lax.rem(k + 1, 3))

        return carry

    lax.fori_loop(0, _NCHUNK, chunk_body, 0)
    pltpu.make_async_copy(
        table_hbm.at[pl.ds(0, _PER_W)],
        out_hbm.at[pl.ds(base, _PER_W)],
        gsem,
    ).wait()


def _add_body(x_ref, p_ref, o_ref):
    o_ref[...] = x_ref[...] + p_ref[...][None]


_ROWS_PER_BLK = 16


def _add_p(emb, p):
    grid = (_BATCH // _ROWS_PER_BLK,)
    return pl.pallas_call(
        _add_body,
        grid=grid,
        in_specs=[
            pl.BlockSpec((_ROWS_PER_BLK, _MAX_LEN, _D), lambda i: (i, 0, 0)),
            pl.BlockSpec((_MAX_LEN, _D), lambda i: (0, 0)),
        ],
        out_specs=pl.BlockSpec((_ROWS_PER_BLK, _MAX_LEN, _D),
                               lambda i: (i, 0, 0)),
        out_shape=jax.ShapeDtypeStruct((_BATCH, _MAX_LEN, _D), jnp.float32),
    )(emb, p)


def kernel(X, table):
    p = _positional()
    xf = X.reshape(-1)
    emb = _gather(xf, table)
    return _add_p(emb.reshape(_BATCH, _MAX_LEN, _D), p)


# speedup vs baseline: 4.7886x; 4.7886x over previous
import functools

import jax
import jax.numpy as jnp
from jax import lax
from jax.experimental import pallas as pl
from jax.experimental.pallas import tpu as pltpu
from jax.experimental.pallas import tpu_sc as plsc

_VOCAB = 1000000
_D = 64
_MAX_LEN = 200
_BATCH = 4096
_B = _BATCH * _MAX_LEN

_NC = 2
_NS = 16
_NW = _NC * _NS
_PER_W = _B // _NW
_C = 200
_NCHUNK = _PER_W // _C
_L = 16
_NBUF = 3


def _positional() -> jax.Array:
    position = jnp.arange(0, _MAX_LEN, dtype=jnp.float32).reshape(-1, 1)
    div = jnp.exp(
        jnp.arange(0, _D, 2, dtype=jnp.float32) / _D
        * -jnp.log(jnp.float32(10000.0))
    )
    p = jnp.zeros((_MAX_LEN, _D), dtype=jnp.float32)
    p = p.at[:, 0::2].set(jnp.sin(position * div))
    p = p.at[:, 1::2].set(jnp.cos(position * div))
    return p


_mesh = plsc.VectorSubcoreMesh(core_axis_name="c", subcore_axis_name="s")


@functools.partial(
    pl.kernel,
    mesh=_mesh,
    out_type=jax.ShapeDtypeStruct((_B, _D), jnp.float32),
    scratch_types=[
        pltpu.VMEM((_PER_W,), jnp.int32),
        pltpu.VMEM((_NBUF, _C, _D), jnp.float32),
        pltpu.VMEM((_MAX_LEN, _D), jnp.float32),
        pltpu.SemaphoreType.DMA((_NBUF,)),
        pltpu.SemaphoreType.DMA((_NBUF,)),
    ],
    compiler_params=pltpu.CompilerParams(use_tc_tiling_on_sc=False),
)
def _embed(x_hbm, table_hbm, p_hbm, out_hbm, idx_all, rows, p_v, gsem, ssem):
    wid = lax.axis_index("s") * _NC + lax.axis_index("c")
    base = wid * _PER_W
    pltpu.sync_copy(p_hbm, p_v)
    pltpu.sync_copy(x_hbm.at[pl.ds(base, _PER_W)], idx_all)

    def gather(k, b):
        pltpu.async_copy(
            table_hbm.at[idx_all.at[pl.ds(k * _C, _C)]], rows.at[b],
            gsem.at[b])

    def gather_wait(k, b):
        pltpu.make_async_copy(
            table_hbm.at[idx_all.at[pl.ds(k * _C, _C)]], rows.at[b],
            gsem.at[b]).wait()

    def store(k, b):
        pltpu.async_copy(
            rows.at[b], out_hbm.at[pl.ds(base + k * _C, _C)], ssem.at[b])

    def store_wait(k, b):
        pltpu.make_async_copy(
            rows.at[b], out_hbm.at[pl.ds(base + k * _C, _C)],
            ssem.at[b]).wait()

    gather(0, 0)

    def chunk_body(k, carry):
        b = lax.rem(k, _NBUF)
        nb = lax.rem(k + 1, _NBUF)

        @pl.when(k + 1 < _NCHUNK)
        def _fire_next():
            @pl.when(k >= _NBUF - 1)
            def _drain_old_store():
                store_wait(k + 1 - _NBUF, nb)

            gather(k + 1, nb)

        gather_wait(k, b)

        def row_body(r, c2):
            for d in range(_D // _L):
                sl = pl.ds(d * _L, _L)
                rows[b, r, sl] = rows[b, r, sl] + p_v[r, sl]
            return c2

        lax.fori_loop(0, _C, row_body, 0, unroll=2)
        store(k, b)
        return carry

    lax.fori_loop(0, _NCHUNK, chunk_body, 0)
    store_wait(_NCHUNK - 2, lax.rem(_NCHUNK - 2, _NBUF))
    store_wait(_NCHUNK - 1, lax.rem(_NCHUNK - 1, _NBUF))


def kernel(X, table):
    p = _positional()
    xf = X.reshape(-1)
    out = _embed(xf, table, p)
    return out.reshape(_BATCH, _MAX_LEN, _D)


# speedup vs baseline: 5.3265x; 1.1123x over previous
import functools

import jax
import jax.numpy as jnp
from jax import lax
from jax.experimental import pallas as pl
from jax.experimental.pallas import tpu as pltpu
from jax.experimental.pallas import tpu_sc as plsc

_VOCAB = 1000000
_D = 64
_MAX_LEN = 200
_BATCH = 4096
_B = _BATCH * _MAX_LEN

_NC = 2
_NS = 16
_NW = _NC * _NS
_PER_W = _B // _NW
_C = 200
_NCHUNK = _PER_W // _C
_L = 16
_NBUF = 3


def _positional() -> jax.Array:
    position = jnp.arange(0, _MAX_LEN, dtype=jnp.float32).reshape(-1, 1)
    div = jnp.exp(
        jnp.arange(0, _D, 2, dtype=jnp.float32) / _D
        * -jnp.log(jnp.float32(10000.0))
    )
    p = jnp.zeros((_MAX_LEN, _D), dtype=jnp.float32)
    p = p.at[:, 0::2].set(jnp.sin(position * div))
    p = p.at[:, 1::2].set(jnp.cos(position * div))
    return p


_mesh = plsc.VectorSubcoreMesh(core_axis_name="c", subcore_axis_name="s")


@functools.partial(
    pl.kernel,
    mesh=_mesh,
    out_type=jax.ShapeDtypeStruct((_B, _D), jnp.float32),
    scratch_types=[
        pltpu.VMEM((_PER_W,), jnp.int32),
        pltpu.VMEM((_NBUF, _C, _D), jnp.float32),
        pltpu.VMEM((_MAX_LEN, _D), jnp.float32),
        pltpu.SemaphoreType.DMA((_NBUF,)),
        pltpu.SemaphoreType.DMA((_NBUF,)),
    ],
    compiler_params=pltpu.CompilerParams(use_tc_tiling_on_sc=False),
)
def _embed(x_hbm, table_hbm, p_hbm, out_hbm, idx_all, rows, p_v, gsem, ssem):
    wid = lax.axis_index("s") * _NC + lax.axis_index("c")
    base = wid * _PER_W
    pltpu.sync_copy(p_hbm, p_v)
    pltpu.sync_copy(x_hbm.at[pl.ds(base, _PER_W)], idx_all)

    def gather(k, b):
        pltpu.async_copy(
            table_hbm.at[idx_all.at[pl.ds(k * _C, _C)]], rows.at[b],
            gsem.at[b])

    def gather_wait(k, b):
        pltpu.make_async_copy(
            table_hbm.at[idx_all.at[pl.ds(k * _C, _C)]], rows.at[b],
            gsem.at[b]).wait()

    def store(k, b):
        pltpu.async_copy(
            rows.at[b], out_hbm.at[pl.ds(base + k * _C, _C)], ssem.at[b])

    def store_wait(k, b):
        pltpu.make_async_copy(
            rows.at[b], out_hbm.at[pl.ds(base + k * _C, _C)],
            ssem.at[b]).wait()

    gather(0, 0)

    def chunk_body(k, carry):
        b = lax.rem(k, _NBUF)
        nb = lax.rem(k + 1, _NBUF)

        @pl.when(k + 1 < _NCHUNK)
        def _fire_next():
            @pl.when(k >= _NBUF - 1)
            def _drain_old_store():
                store_wait(k + 1 - _NBUF, nb)

            gather(k + 1, nb)

        gather_wait(k, b)

        def row_body(r, c2):
            for d in range(_D // _L):
                sl = pl.ds(d * _L, _L)
                plsc.addupdate(rows.at[b, r, sl], p_v[r, sl])
            return c2

        lax.fori_loop(0, _C, row_body, 0, unroll=2)
        store(k, b)
        return carry

    lax.fori_loop(0, _NCHUNK, chunk_body, 0)
    store_wait(_NCHUNK - 2, lax.rem(_NCHUNK - 2, _NBUF))
    store_wait(_NCHUNK - 1, lax.rem(_NCHUNK - 1, _NBUF))


def kernel(X, table):
    p = _positional()
    xf = X.reshape(-1)
    out = _embed(xf, table, p)
    return out.reshape(_BATCH, _MAX_LEN, _D)
